# baseline (speedup 1.0000x reference)
import functools

import jax
import jax.numpy as jnp
from jax import lax
from jax.experimental import pallas as pl
from jax.experimental.pallas import tpu as pltpu
from jax.experimental.pallas import tpu_sc as plsc

N = 10000
E = 320000
R = 8
NSEG = N * R
SLABW = 16
NSLAB = 8
NC, NS = 2, 16
EPW = E // NS
BATCH = 250
CHUNKS = EPW // BATCH
ROWS_PT = NSEG // NS
ZROWS = 125


def _sc_segsum(h8, idx_all, seg_r, with_counts):
    mesh = plsc.VectorSubcoreMesh(core_axis_name="c", subcore_axis_name="s")
    out_type = [jax.ShapeDtypeStruct((NSEG, NSLAB * SLABW), jnp.float32)]
    if with_counts:
        out_type.append(jax.ShapeDtypeStruct((NC, NSEG, SLABW), jnp.float32))

    @functools.partial(
        pl.kernel,
        out_type=tuple(out_type),
        mesh=mesh,
        scratch_types=[
            pltpu.VMEM((CHUNKS, BATCH), jnp.int32),
            pltpu.VMEM((CHUNKS, BATCH), jnp.int32),
            pltpu.VMEM((BATCH, SLABW), jnp.float32),
            pltpu.VMEM((BATCH, SLABW), jnp.float32),
            pltpu.VMEM((ZROWS, SLABW), jnp.float32),
            pltpu.VMEM_SHARED((NSEG, SLABW), jnp.float32),
            pltpu.SemaphoreType.DMA,
            pltpu.SemaphoreType.DMA,
            pltpu.SemaphoreType.DMA,
            pltpu.SemaphoreType.DMA,
        ],
        compiler_params=pltpu.CompilerParams(use_tc_tiling_on_sc=False),
    )
    def k(h_hbm, idx_hbm, seg_hbm, *refs):
        if with_counts:
            (s_hbm, cnt_hbm, idx_v, seg_v, rows0_v, rows1_v, zeros_v, accum,
             sem0, sem1, ssem0, ssem1) = refs
        else:
            (s_hbm, idx_v, seg_v, rows0_v, rows1_v, zeros_v, accum,
             sem0, sem1, ssem0, ssem1) = refs
            cnt_hbm = None
        c = lax.axis_index("c")
        s = lax.axis_index("s")

        pltpu.sync_copy(seg_hbm.at[s], seg_v)

        def zfill(i, _):
            zeros_v[i] = jnp.zeros((SLABW,), jnp.float32)
            return _
        lax.fori_loop(0, ZROWS, zfill, None)

        def zero_accum():
            def zero_blk(z, _):
                pltpu.sync_copy(
                    zeros_v, accum.at[pl.ds(s * ROWS_PT + z * ZROWS, ZROWS)])
                return _
            lax.fori_loop(0, ROWS_PT // ZROWS, zero_blk, None)
            plsc.subcore_barrier()

        if with_counts:
            def ofill(i, _):
                rows0_v[i] = jnp.ones((SLABW,), jnp.float32)
                return _
            lax.fori_loop(0, BATCH, ofill, None)
            zero_accum()

            def cnt_blk(j, _):
                pltpu.sync_copy(
                    rows0_v, accum.at[seg_v.at[c * (CHUNKS // NC) + j]],
                    add=True)
                return _
            lax.fori_loop(0, CHUNKS // NC, cnt_blk, None)
            plsc.subcore_barrier()
            pltpu.sync_copy(
                accum.at[pl.ds(s * ROWS_PT, ROWS_PT)],
                cnt_hbm.at[c].at[pl.ds(s * ROWS_PT, ROWS_PT)])
            plsc.subcore_barrier()

        def gather(jc, buf, sem, slab):
            pltpu.async_copy(h_hbm.at[idx_v.at[jc]], buf, sem)

        def gwait(buf, sem):
            pltpu.make_async_copy(h_hbm.at[idx_v.at[0]], buf, sem).wait()

        def scat(jc, buf, sem):
            pltpu.async_copy(buf, accum.at[seg_v.at[jc]], sem, add=True)

        def swait(buf, sem):
            pltpu.make_async_copy(buf, accum.at[seg_v.at[0]], sem).wait()

        for jslab in range(NSLAB // NC):
            slab = c * (NSLAB // NC) + jslab
            pltpu.sync_copy(idx_hbm.at[slab].at[s], idx_v)
            zero_accum()

            gather(0, rows0_v, sem0, slab)
            gather(1, rows1_v, sem1, slab)

            def edge_pair(i, _):
                j0 = 2 * i
                gwait(rows0_v, sem0)
                scat(j0, rows0_v, ssem0)
                gwait(rows1_v, sem1)
                scat(j0 + 1, rows1_v, ssem1)
                swait(rows0_v, ssem0)
                gather(lax.min(j0 + 2, CHUNKS - 1), rows0_v, sem0, slab)
                swait(rows1_v, ssem1)
                gather(lax.min(j0 + 3, CHUNKS - 1), rows1_v, sem1, slab)
                return _
            lax.fori_loop(0, CHUNKS // 2, edge_pair, None)
            gwait(rows0_v, sem0)
            gwait(rows1_v, sem1)
            plsc.subcore_barrier()

            pltpu.sync_copy(
                accum.at[pl.ds(s * ROWS_PT, ROWS_PT)],
                s_hbm.at[pl.ds(s * ROWS_PT, ROWS_PT),
                         pl.ds(SLABW * slab, SLABW)])
            plsc.subcore_barrier()

    return k(h8, idx_all, seg_r)


def _prelu(x, a):
    return jnp.where(x >= 0, x, a * x)


def kernel(x, num_x, W_num, b_num, a_in, comp1, bases1, root1, bias1, a1, comp2, bases2, root2, bias2, a2, comp3, bases3, root3, bias3, edge_index, edge_type):
    src = edge_index[0]
    dst = edge_index[1]
    seg = dst * R + edge_type
    idx_all = (src * NSLAB)[None, :] + jnp.arange(NSLAB, dtype=jnp.int32)[:, None]
    idx_all = idx_all.reshape(NSLAB, NS, CHUNKS, BATCH)
    seg_r = seg.reshape(NS, CHUNKS, BATCH)

    h = _prelu(num_x @ W_num + b_num, a_in) + x

    def layer(h, comp, bases, root, bias, with_counts, inv):
        res = _sc_segsum(h.reshape(N * NSLAB, SLABW), idx_all, seg_r,
                         with_counts)
        if with_counts:
            S, cnt16 = res
            cnt = cnt16[0, :, 0] + cnt16[1, :, 0]
            inv = 1.0 / jnp.maximum(cnt, 1.0)
        else:
            (S,) = res
        T = S * inv[:, None]
        W = jnp.einsum("rb,bio->rio", comp, bases)
        agg = jnp.einsum("nrd,rdo->no", T.reshape(N, R, -1), W)
        return agg + h @ root + bias, inv

    h1, inv = layer(h, comp1, bases1, root1, bias1, True, None)
    h = _prelu(h1, a1)
    h2, _ = layer(h, comp2, bases2, root2, bias2, False, inv)
    h = _prelu(h2, a2)
    h3, _ = layer(h, comp3, bases3, root3, bias3, False, inv)
    return jax.nn.log_softmax(h3, axis=1)


# speedup vs baseline: 1.0561x; 1.0561x over previous
import functools

import jax
import jax.numpy as jnp
from jax import lax
from jax.experimental import pallas as pl
from jax.experimental.pallas import tpu as pltpu
from jax.experimental.pallas import tpu_sc as plsc

N = 10000
E = 320000
R = 8
NSEG = N * R
SLABW = 16
NSLAB = 8
NC, NS = 2, 16
EPW = E // NS
BATCH = 250
CHUNKS = EPW // BATCH
ROWS_PT = NSEG // NS
ZROWS = 125
NBUF = 2


def _sc_segsum(h8, idx_all, seg_r, with_counts):
    mesh = plsc.VectorSubcoreMesh(core_axis_name="c", subcore_axis_name="s")
    out_type = [jax.ShapeDtypeStruct((NSEG, NSLAB * SLABW), jnp.float32)]
    if with_counts:
        out_type.append(jax.ShapeDtypeStruct((NC, NSEG, SLABW), jnp.float32))

    @functools.partial(
        pl.kernel,
        out_type=tuple(out_type),
        mesh=mesh,
        scratch_types=[
            pltpu.VMEM((CHUNKS, BATCH), jnp.int32),
            pltpu.VMEM((CHUNKS, BATCH), jnp.int32),
            pltpu.VMEM((NBUF, BATCH, SLABW), jnp.float32),
            pltpu.VMEM((ZROWS, SLABW), jnp.float32),
            pltpu.VMEM_SHARED((NSEG, SLABW), jnp.float32),
            [pltpu.SemaphoreType.DMA] * NBUF,
            [pltpu.SemaphoreType.DMA] * NBUF,
            pltpu.SemaphoreType.DMA,
        ],
        compiler_params=pltpu.CompilerParams(use_tc_tiling_on_sc=False),
    )
    def k(h_hbm, idx_hbm, seg_hbm, *refs):
        if with_counts:
            (s_hbm, cnt_hbm, idx_v, seg_v, rows_v, zeros_v, accum,
             gsem, ssem, zsem) = refs
        else:
            (s_hbm, idx_v, seg_v, rows_v, zeros_v, accum,
             gsem, ssem, zsem) = refs
            cnt_hbm = None
        c = lax.axis_index("c")
        s = lax.axis_index("s")

        pltpu.sync_copy(seg_hbm.at[s], seg_v)

        def zfill(i, _):
            zeros_v[i] = jnp.zeros((SLABW,), jnp.float32)
            return _
        lax.fori_loop(0, ZROWS, zfill, None)

        def zero_accum():
            def zero_blk(z, _):
                pltpu.async_copy(
                    zeros_v, accum.at[pl.ds(s * ROWS_PT + z * ZROWS, ZROWS)],
                    zsem)
                return _
            lax.fori_loop(0, ROWS_PT // ZROWS, zero_blk, None)

            def zero_drain(z, _):
                pltpu.make_async_copy(
                    zeros_v, accum.at[pl.ds(s * ROWS_PT, ZROWS)], zsem).wait()
                return _
            lax.fori_loop(0, ROWS_PT // ZROWS, zero_drain, None)
            plsc.subcore_barrier()

        if with_counts:
            def ofill(i, _):
                rows_v[0, i] = jnp.ones((SLABW,), jnp.float32)
                return _
            lax.fori_loop(0, BATCH, ofill, None)
            zero_accum()

            def cnt_blk(j, _):
                pltpu.async_copy(
                    rows_v.at[0], accum.at[seg_v.at[c * (CHUNKS // NC) + j]],
                    ssem[0], add=True)
                return _
            lax.fori_loop(0, CHUNKS // NC, cnt_blk, None)

            def cnt_drain(j, _):
                pltpu.make_async_copy(
                    rows_v.at[0], accum.at[seg_v.at[0]], ssem[0]).wait()
                return _
            lax.fori_loop(0, CHUNKS // NC, cnt_drain, None)
            plsc.subcore_barrier()
            pltpu.sync_copy(
                accum.at[pl.ds(s * ROWS_PT, ROWS_PT)],
                cnt_hbm.at[c].at[pl.ds(s * ROWS_PT, ROWS_PT)])
            plsc.subcore_barrier()

        def gather(jc, k):
            pltpu.async_copy(h_hbm.at[idx_v.at[jc]], rows_v.at[k], gsem[k])

        def gwait(k):
            pltpu.make_async_copy(
                h_hbm.at[idx_v.at[0]], rows_v.at[k], gsem[k]).wait()

        def scat(jc, k):
            pltpu.async_copy(
                rows_v.at[k], accum.at[seg_v.at[jc]], ssem[k], add=True)

        def swait(k):
            pltpu.make_async_copy(
                rows_v.at[k], accum.at[seg_v.at[0]], ssem[k]).wait()

        def slab_body(jslab, _):
            slab = c * (NSLAB // NC) + jslab
            pltpu.sync_copy(idx_hbm.at[slab].at[s], idx_v)
            zero_accum()

            gather(0, 0)

            def edge_pair(i, _):
                j0 = 2 * i
                gather(j0 + 1, 1)
                gwait(0)
                pltpu.sync_copy(rows_v.at[0], accum.at[seg_v.at[j0]], add=True)
                gather(lax.min(j0 + 2, CHUNKS - 1), 0)
                gwait(1)
                pltpu.sync_copy(
                    rows_v.at[1], accum.at[seg_v.at[j0 + 1]], add=True)
                return _
            lax.fori_loop(0, CHUNKS // 2, edge_pair, None)
            gwait(0)
            plsc.subcore_barrier()

            pltpu.sync_copy(
                accum.at[pl.ds(s * ROWS_PT, ROWS_PT)],
                s_hbm.at[pl.ds(s * ROWS_PT, ROWS_PT),
                         pl.ds(SLABW * slab, SLABW)])
            plsc.subcore_barrier()
            return _

        lax.fori_loop(0, NSLAB // NC, slab_body, None)

    return k(h8, idx_all, seg_r)


def _prelu(x, a):
    return jnp.where(x >= 0, x, a * x)


def kernel(x, num_x, W_num, b_num, a_in, comp1, bases1, root1, bias1, a1, comp2, bases2, root2, bias2, a2, comp3, bases3, root3, bias3, edge_index, edge_type):
    src = edge_index[0]
    dst = edge_index[1]
    seg = dst * R + edge_type
    idx_all = (src * NSLAB)[None, :] + jnp.arange(NSLAB, dtype=jnp.int32)[:, None]
    idx_all = idx_all.reshape(NSLAB, NS, CHUNKS, BATCH)
    seg_r = seg.reshape(NS, CHUNKS, BATCH)

    h = _prelu(num_x @ W_num + b_num, a_in) + x

    def layer(h, comp, bases, root, bias, with_counts, inv):
        res = _sc_segsum(h.reshape(N * NSLAB, SLABW), idx_all, seg_r,
                         with_counts)
        if with_counts:
            S, cnt16 = res
            cnt = cnt16[0, :, 0] + cnt16[1, :, 0]
            inv = 1.0 / jnp.maximum(cnt, 1.0)
        else:
            (S,) = res
        T = S * inv[:, None]
        W = jnp.einsum("rb,bio->rio", comp, bases)
        agg = jnp.einsum("nrd,rdo->no", T.reshape(N, R, -1), W)
        return agg + h @ root + bias, inv

    h1, inv = layer(h, comp1, bases1, root1, bias1, True, None)
    h = _prelu(h1, a1)
    h2, _ = layer(h, comp2, bases2, root2, bias2, False, inv)
    h = _prelu(h2, a2)
    h3, _ = layer(h, comp3, bases3, root3, bias3, False, inv)
    return jax.nn.log_softmax(h3, axis=1)


# speedup vs baseline: 1.0925x; 1.0345x over previous
import functools

import jax
import jax.numpy as jnp
from jax import lax
from jax.experimental import pallas as pl
from jax.experimental.pallas import tpu as pltpu
from jax.experimental.pallas import tpu_sc as plsc

N = 10000
E = 320000
R = 8
NSEG = N * R
SLABW = 16
NSLAB = 8
NC, NS = 2, 16
EPW = E // NS
BATCH = 250
CHUNKS = EPW // BATCH
ROWS_PT = NSEG // NS
ZROWS = 125
NBUF = 2


def _sc_segsum(h8, idx_all, seg_r, with_counts):
    mesh = plsc.VectorSubcoreMesh(core_axis_name="c", subcore_axis_name="s")
    out_type = [jax.ShapeDtypeStruct((NSEG, NSLAB * SLABW), jnp.float32)]
    if with_counts:
        out_type.append(jax.ShapeDtypeStruct((NC, NSEG, SLABW), jnp.float32))

    @functools.partial(
        pl.kernel,
        out_type=tuple(out_type),
        mesh=mesh,
        scratch_types=[
            pltpu.VMEM((CHUNKS, BATCH), jnp.int32),
            pltpu.VMEM((CHUNKS, BATCH), jnp.int32),
            pltpu.VMEM((NBUF, BATCH, SLABW), jnp.float32),
            pltpu.VMEM((ZROWS, SLABW), jnp.float32),
            pltpu.VMEM_SHARED((NSEG, SLABW), jnp.float32),
            [pltpu.SemaphoreType.DMA] * NBUF,
            [pltpu.SemaphoreType.DMA] * NBUF,
            pltpu.SemaphoreType.DMA,
        ],
        compiler_params=pltpu.CompilerParams(use_tc_tiling_on_sc=False),
    )
    def k(h_hbm, idx_hbm, seg_hbm, *refs):
        if with_counts:
            (s_hbm, cnt_hbm, idx_v, seg_v, rows_v, zeros_v, accum,
             gsem, ssem, zsem) = refs
        else:
            (s_hbm, idx_v, seg_v, rows_v, zeros_v, accum,
             gsem, ssem, zsem) = refs
            cnt_hbm = None
        c = lax.axis_index("c")
        s = lax.axis_index("s")

        pltpu.sync_copy(seg_hbm.at[s], seg_v)

        def zfill(i, _):
            zeros_v[i] = jnp.zeros((SLABW,), jnp.float32)
            return _
        lax.fori_loop(0, ZROWS, zfill, None)

        def zero_accum():
            def zero_blk(z, _):
                pltpu.async_copy(
                    zeros_v, accum.at[pl.ds(s * ROWS_PT + z * ZROWS, ZROWS)],
                    zsem)
                return _
            lax.fori_loop(0, ROWS_PT // ZROWS, zero_blk, None)

            def zero_drain(z, _):
                pltpu.make_async_copy(
                    zeros_v, accum.at[pl.ds(s * ROWS_PT, ZROWS)], zsem).wait()
                return _
            lax.fori_loop(0, ROWS_PT // ZROWS, zero_drain, None)
            plsc.subcore_barrier()

        if with_counts:
            def ofill(i, _):
                rows_v[0, i] = jnp.ones((SLABW,), jnp.float32)
                return _
            lax.fori_loop(0, BATCH, ofill, None)
            zero_accum()

            def cnt_blk(j, _):
                pltpu.async_copy(
                    rows_v.at[0], accum.at[seg_v.at[c * (CHUNKS // NC) + j]],
                    ssem[0], add=True)
                return _
            lax.fori_loop(0, CHUNKS // NC, cnt_blk, None)

            def cnt_drain(j, _):
                pltpu.make_async_copy(
                    rows_v.at[0], accum.at[seg_v.at[0]], ssem[0]).wait()
                return _
            lax.fori_loop(0, CHUNKS // NC, cnt_drain, None)
            plsc.subcore_barrier()
            pltpu.sync_copy(
                accum.at[pl.ds(s * ROWS_PT, ROWS_PT)],
                cnt_hbm.at[c].at[pl.ds(s * ROWS_PT, ROWS_PT)])
            plsc.subcore_barrier()

        def gather(jc, k):
            pltpu.async_copy(h_hbm.at[idx_v.at[jc]], rows_v.at[k], gsem[k])

        def gwait(k):
            pltpu.make_async_copy(
                h_hbm.at[idx_v.at[0]], rows_v.at[k], gsem[k]).wait()

        def scat(jc, k):
            pltpu.async_copy(
                rows_v.at[k], accum.at[seg_v.at[jc]], ssem[k], add=True)

        def swait(k):
            pltpu.make_async_copy(
                rows_v.at[k], accum.at[seg_v.at[0]], ssem[k]).wait()

        def slab_body(jslab, _):
            slab = c * (NSLAB // NC) + jslab
            pltpu.sync_copy(idx_hbm.at[slab].at[s], idx_v)
            zero_accum()

            gather(0, 0)

            def edge_pair(i, _):
                j0 = 2 * i
                gather(j0 + 1, 1)
                gwait(0)
                pltpu.sync_copy(rows_v.at[0], accum.at[seg_v.at[j0]], add=True)
                gather(lax.min(j0 + 2, CHUNKS - 1), 0)
                gwait(1)
                pltpu.sync_copy(
                    rows_v.at[1], accum.at[seg_v.at[j0 + 1]], add=True)
                return _
            lax.fori_loop(0, CHUNKS // 2, edge_pair, None)
            gwait(0)
            plsc.subcore_barrier()

            pltpu.sync_copy(
                accum.at[pl.ds(s * ROWS_PT, ROWS_PT)],
                s_hbm.at[pl.ds(s * ROWS_PT, ROWS_PT),
                         pl.ds(SLABW * slab, SLABW)])
            plsc.subcore_barrier()
            return _

        lax.fori_loop(0, NSLAB // NC, slab_body, None)

    return k(h8, idx_all, seg_r)


def _prelu(x, a):
    return jnp.where(x >= 0, x, a * x)


def _tc_dense(Tp, h, W, root, bias, act_a, final):
    DO = W.shape[-1]
    BN = 1000

    def body(t_ref, h_ref, w_ref, r_ref, b_ref, *rest):
        if final:
            (o_ref,) = rest
        else:
            a_ref, o_ref = rest
        acc = jnp.dot(h_ref[...], r_ref[...],
                      preferred_element_type=jnp.float32)
        for r in range(R):
            acc += jnp.dot(t_ref[r], w_ref[r],
                           preferred_element_type=jnp.float32)
        acc += b_ref[...]
        if final:
            m = jnp.max(acc, axis=-1, keepdims=True)
            e = jnp.exp(acc - m)
            lse = jnp.log(jnp.sum(e, axis=-1, keepdims=True)) + m
            o_ref[...] = acc - lse
        else:
            a = a_ref[...]
            o_ref[...] = jnp.where(acc >= 0, acc, a * acc)

    in_specs = [
        pl.BlockSpec((R, BN, 128), lambda i: (0, i, 0)),
        pl.BlockSpec((BN, 128), lambda i: (i, 0)),
        pl.BlockSpec((R, 128, DO), lambda i: (0, 0, 0)),
        pl.BlockSpec((128, DO), lambda i: (0, 0)),
        pl.BlockSpec((1, DO), lambda i: (0, 0)),
    ]
    args = [Tp, h, W, root, bias]
    if not final:
        in_specs.append(pl.BlockSpec((1, DO), lambda i: (0, 0)))
        args.append(act_a)
    return pl.pallas_call(
        body,
        grid=(N // BN,),
        in_specs=in_specs,
        out_specs=pl.BlockSpec((BN, DO), lambda i: (i, 0)),
        out_shape=jax.ShapeDtypeStruct((N, DO), jnp.float32),
    )(*args)


def kernel(x, num_x, W_num, b_num, a_in, comp1, bases1, root1, bias1, a1, comp2, bases2, root2, bias2, a2, comp3, bases3, root3, bias3, edge_index, edge_type):
    src = edge_index[0]
    dst = edge_index[1]
    seg = edge_type * N + dst
    idx_all = (src * NSLAB)[None, :] + jnp.arange(NSLAB, dtype=jnp.int32)[:, None]
    idx_all = idx_all.reshape(NSLAB, NS, CHUNKS, BATCH)
    seg_r = seg.reshape(NS, CHUNKS, BATCH)

    h = _prelu(num_x * W_num[0] + b_num, a_in) + x

    def layer(h, comp, bases, root, bias, act_a, with_counts, final, inv):
        res = _sc_segsum(h.reshape(N * NSLAB, SLABW), idx_all, seg_r,
                         with_counts)
        if with_counts:
            S, cnt16 = res
            cnt = cnt16[0, :, 0] + cnt16[1, :, 0]
            inv = 1.0 / jnp.maximum(cnt, 1.0)
        else:
            (S,) = res
        Tp = S.reshape(R, N, -1) * inv.reshape(R, N)[:, :, None]
        W = jnp.einsum("rb,bio->rio", comp, bases)
        act2 = None if final else act_a[None, :]
        return _tc_dense(Tp, h, W, root, bias[None, :], act2, final), inv

    h, inv = layer(h, comp1, bases1, root1, bias1, a1, True, False, None)
    h, _ = layer(h, comp2, bases2, root2, bias2, a2, False, False, inv)
    out, _ = layer(h, comp3, bases3, root3, bias3, None, False, True, inv)
    return out


# speedup vs baseline: 1.1281x; 1.0326x over previous
import functools

import jax
import jax.numpy as jnp
from jax import lax
from jax.experimental import pallas as pl
from jax.experimental.pallas import tpu as pltpu
from jax.experimental.pallas import tpu_sc as plsc

N = 10000
E = 320000
R = 8
NSEG = N * R
SLABW = 16
NSLAB = 8
NC, NS = 2, 16
EPW = E // NS
BATCH = 250
CHUNKS = EPW // BATCH
ROWS_PT = NSEG // NS
ZROWS = 125
NBUF = 2


def _sc_segsum(h8, idx_all, seg_r, with_counts):
    mesh = plsc.VectorSubcoreMesh(core_axis_name="c", subcore_axis_name="s")
    out_type = [jax.ShapeDtypeStruct((NSEG, NSLAB * SLABW), jnp.float32)]
    if with_counts:
        out_type.append(jax.ShapeDtypeStruct((NC, NSEG, SLABW), jnp.float32))

    @functools.partial(
        pl.kernel,
        out_type=tuple(out_type),
        mesh=mesh,
        scratch_types=[
            pltpu.VMEM((CHUNKS, BATCH), jnp.int32),
            pltpu.VMEM((CHUNKS, BATCH), jnp.int32),
            pltpu.VMEM((NBUF, BATCH, SLABW), jnp.float32),
            pltpu.VMEM((ZROWS, SLABW), jnp.float32),
            pltpu.VMEM_SHARED((NSEG, SLABW), jnp.float32),
            [pltpu.SemaphoreType.DMA] * NBUF,
            [pltpu.SemaphoreType.DMA] * NBUF,
            pltpu.SemaphoreType.DMA,
        ],
        compiler_params=pltpu.CompilerParams(use_tc_tiling_on_sc=False),
    )
    def k(h_hbm, idx_hbm, seg_hbm, *refs):
        if with_counts:
            (s_hbm, cnt_hbm, idx_v, seg_v, rows_v, zeros_v, accum,
             gsem, ssem, zsem) = refs
        else:
            (s_hbm, idx_v, seg_v, rows_v, zeros_v, accum,
             gsem, ssem, zsem) = refs
            cnt_hbm = None
        c = lax.axis_index("c")
        s = lax.axis_index("s")

        pltpu.sync_copy(seg_hbm.at[s], seg_v)

        def zfill(i, _):
            zeros_v[i] = jnp.zeros((SLABW,), jnp.float32)
            return _
        lax.fori_loop(0, ZROWS, zfill, None)

        def zero_accum():
            def zero_blk(z, _):
                pltpu.async_copy(
                    zeros_v, accum.at[pl.ds(s * ROWS_PT + z * ZROWS, ZROWS)],
                    zsem)
                return _
            lax.fori_loop(0, ROWS_PT // ZROWS, zero_blk, None)

            def zero_drain(z, _):
                pltpu.make_async_copy(
                    zeros_v, accum.at[pl.ds(s * ROWS_PT, ZROWS)], zsem).wait()
                return _
            lax.fori_loop(0, ROWS_PT // ZROWS, zero_drain, None)
            plsc.subcore_barrier()

        if with_counts:
            def ofill(i, _):
                rows_v[0, i] = jnp.ones((SLABW,), jnp.float32)
                return _
            lax.fori_loop(0, BATCH, ofill, None)
            zero_accum()

            def cnt_blk(j, _):
                pltpu.async_copy(
                    rows_v.at[0], accum.at[seg_v.at[c * (CHUNKS // NC) + j]],
                    ssem[0], add=True)
                return _
            lax.fori_loop(0, CHUNKS // NC, cnt_blk, None)

            def cnt_drain(j, _):
                pltpu.make_async_copy(
                    rows_v.at[0], accum.at[seg_v.at[0]], ssem[0]).wait()
                return _
            lax.fori_loop(0, CHUNKS // NC, cnt_drain, None)
            plsc.subcore_barrier()
            pltpu.sync_copy(
                accum.at[pl.ds(s * ROWS_PT, ROWS_PT)],
                cnt_hbm.at[c].at[pl.ds(s * ROWS_PT, ROWS_PT)])
            plsc.subcore_barrier()

        def gather(jc, k):
            pltpu.async_copy(h_hbm.at[idx_v.at[jc]], rows_v.at[k], gsem[k])

        def gwait(k):
            pltpu.make_async_copy(
                h_hbm.at[idx_v.at[0]], rows_v.at[k], gsem[k]).wait()

        def scat(jc, k):
            pltpu.async_copy(
                rows_v.at[k], accum.at[seg_v.at[jc]], ssem[k], add=True)

        def swait(k):
            pltpu.make_async_copy(
                rows_v.at[k], accum.at[seg_v.at[0]], ssem[k]).wait()

        def slab_body(jslab, _):
            slab = c * (NSLAB // NC) + jslab
            pltpu.sync_copy(idx_hbm.at[slab].at[s], idx_v)
            zero_accum()

            gather(0, 0)

            def edge_pair(i, _):
                j0 = 2 * i
                gather(j0 + 1, 1)
                gwait(0)
                pltpu.sync_copy(rows_v.at[0], accum.at[seg_v.at[j0]], add=True)
                gather(lax.min(j0 + 2, CHUNKS - 1), 0)
                gwait(1)
                pltpu.sync_copy(
                    rows_v.at[1], accum.at[seg_v.at[j0 + 1]], add=True)
                return _
            lax.fori_loop(0, CHUNKS // 2, edge_pair, None)
            gwait(0)
            plsc.subcore_barrier()

            pltpu.sync_copy(
                accum.at[pl.ds(s * ROWS_PT, ROWS_PT)],
                s_hbm.at[pl.ds(s * ROWS_PT, ROWS_PT),
                         pl.ds(SLABW * slab, SLABW)])
            plsc.subcore_barrier()
            return _

        lax.fori_loop(0, NSLAB // NC, slab_body, None)

    return k(h8, idx_all, seg_r)


def _prelu(x, a):
    return jnp.where(x >= 0, x, a * x)


def _tc_dense(S, inv, h, W, root, bias, act_a, final):
    DO = W.shape[-1]
    BN = 1000

    def body(t_ref, i_ref, h_ref, w_ref, r_ref, b_ref, *rest):
        if final:
            (o_ref,) = rest
        else:
            a_ref, o_ref = rest
        acc = jnp.dot(h_ref[...], r_ref[...],
                      preferred_element_type=jnp.float32)
        for r in range(R):
            acc += jnp.dot(t_ref[r] * i_ref[r], w_ref[r],
                           preferred_element_type=jnp.float32)
        acc += b_ref[...]
        if final:
            m = jnp.max(acc, axis=-1, keepdims=True)
            e = jnp.exp(acc - m)
            lse = jnp.log(jnp.sum(e, axis=-1, keepdims=True)) + m
            o_ref[...] = acc - lse
        else:
            a = a_ref[...]
            o_ref[...] = jnp.where(acc >= 0, acc, a * acc)

    in_specs = [
        pl.BlockSpec((R, BN, 128), lambda i: (0, i, 0)),
        pl.BlockSpec((R, BN, 1), lambda i: (0, i, 0)),
        pl.BlockSpec((BN, 128), lambda i: (i, 0)),
        pl.BlockSpec((R, 128, DO), lambda i: (0, 0, 0)),
        pl.BlockSpec((128, DO), lambda i: (0, 0)),
        pl.BlockSpec((1, DO), lambda i: (0, 0)),
    ]
    args = [S, inv, h, W, root, bias]
    if not final:
        in_specs.append(pl.BlockSpec((1, DO), lambda i: (0, 0)))
        args.append(act_a)
    return pl.pallas_call(
        body,
        grid=(N // BN,),
        in_specs=in_specs,
        out_specs=pl.BlockSpec((BN, DO), lambda i: (i, 0)),
        out_shape=jax.ShapeDtypeStruct((N, DO), jnp.float32),
    )(*args)


def kernel(x, num_x, W_num, b_num, a_in, comp1, bases1, root1, bias1, a1, comp2, bases2, root2, bias2, a2, comp3, bases3, root3, bias3, edge_index, edge_type):
    src = edge_index[0]
    dst = edge_index[1]
    seg = edge_type * N + dst
    idx_all = (src * NSLAB)[None, :] + jnp.arange(NSLAB, dtype=jnp.int32)[:, None]
    idx_all = idx_all.reshape(NSLAB, NS, CHUNKS, BATCH)
    seg_r = seg.reshape(NS, CHUNKS, BATCH)

    h = _prelu(num_x * W_num[0] + b_num, a_in) + x

    def layer(h, comp, bases, root, bias, act_a, with_counts, final, inv):
        res = _sc_segsum(h.reshape(N * NSLAB, SLABW), idx_all, seg_r,
                         with_counts)
        if with_counts:
            S, cnt16 = res
            cnt = cnt16[0, :, 0] + cnt16[1, :, 0]
            inv = 1.0 / jnp.maximum(cnt, 1.0)
        else:
            (S,) = res
        W = jnp.einsum("rb,bio->rio", comp, bases)
        act2 = None if final else act_a[None, :]
        out = _tc_dense(S.reshape(R, N, -1), inv.reshape(R, N, 1), h, W,
                        root, bias[None, :], act2, final)
        return out, inv

    h, inv = layer(h, comp1, bases1, root1, bias1, a1, True, False, None)
    h, _ = layer(h, comp2, bases2, root2, bias2, a2, False, False, inv)
    out, _ = layer(h, comp3, bases3, root3, bias3, None, False, True, inv)
    return out
